# baseline (speedup 1.0000x reference)
import functools

import jax
import jax.numpy as jnp
from jax import lax
from jax.experimental import pallas as pl
from jax.experimental.pallas import tpu as pltpu
from jax.experimental.pallas import tpu_sc as plsc

N = 10000
D = 128
E = 320000

NC = 2
NS = 16
NW = NC * NS
CHUNK = 128
NCHUNK = 80
IB = 16
NGROUP = NCHUNK // IB
E_PAD = NW * NCHUNK * CHUNK
N_PAD = 10240
STRIPE = N_PAD // NS
ROWS_PER_TILE_COPY = 128


def _sc_aggregate_kernel(x_hbm, src_hbm, dst_hbm, sums_hbm, counts_hbm,
                         src_v, dst_v, counts_v, gbuf0, gbuf1, sums_acc,
                         sem0, sem1):
    c = lax.axis_index("c")
    s = lax.axis_index("s")
    wid = s * NC + c

    zeros16 = jnp.zeros((16,), jnp.float32)

    def _zrow(i, _):
        for k in range(D // 16):
            gbuf1[i, pl.ds(k * 16, 16)] = zeros16
        return 0
    lax.fori_loop(0, ROWS_PER_TILE_COPY, _zrow, 0)

    def _zcnt(i, _):
        counts_v[pl.ds(i * 16, 16)] = zeros16
        return 0
    lax.fori_loop(0, N_PAD // 16, _zcnt, 0)

    for k in range(STRIPE // ROWS_PER_TILE_COPY):
        pltpu.sync_copy(
            gbuf1,
            sums_acc.at[pl.ds(s * STRIPE + k * ROWS_PER_TILE_COPY,
                              ROWS_PER_TILE_COPY)])

    plsc.subcore_barrier()

    ones16 = jnp.ones((16,), jnp.float32)

    def _half(m, gbuf, sem, refill):
        pltpu.make_async_copy(x_hbm.at[src_v.at[m]], gbuf, sem).wait()
        if refill:
            pltpu.async_copy(x_hbm.at[src_v.at[m + 2]], gbuf, sem)

    def _group(g, _):
        pltpu.sync_copy(src_hbm.at[wid, pl.ds(g * IB, IB)], src_v)
        pltpu.sync_copy(dst_hbm.at[wid, pl.ds(g * IB, IB)], dst_v)

        def _chunk(m, _):
            pltpu.async_copy(x_hbm.at[src_v.at[m]], gbuf0, sem0).wait()
            return 0

        lax.fori_loop(0, IB, _chunk, 0)
        return 0

    lax.fori_loop(0, NGROUP, _group, 0)

    plsc.subcore_barrier()

    pltpu.sync_copy(sums_acc.at[pl.ds(s * STRIPE, STRIPE)],
                    sums_hbm.at[c, pl.ds(s * STRIPE, STRIPE)])
    pltpu.sync_copy(counts_v, counts_hbm.at[wid])


def _sc_aggregate(x, src_r, dst_r):
    mesh = plsc.VectorSubcoreMesh(core_axis_name="c", subcore_axis_name="s")
    return pl.kernel(
        _sc_aggregate_kernel,
        out_type=(
            jax.ShapeDtypeStruct((NC, N_PAD, D), jnp.float32),
            jax.ShapeDtypeStruct((NW, N_PAD), jnp.float32),
        ),
        mesh=mesh,
        scratch_types=[
            pltpu.VMEM((IB, CHUNK), jnp.int32),
            pltpu.VMEM((IB, CHUNK), jnp.int32),
            pltpu.VMEM((N_PAD,), jnp.float32),
            pltpu.VMEM((CHUNK, D), jnp.float32),
            pltpu.VMEM((CHUNK, D), jnp.float32),
            pltpu.VMEM_SHARED((N_PAD, D), jnp.float32),
            pltpu.SemaphoreType.DMA,
            pltpu.SemaphoreType.DMA,
        ],
        compiler_params=pltpu.CompilerParams(needs_layout_passes=False),
    )(x, src_r, dst_r)


def _tc_finish_kernel(sums_ref, counts_ref, x_ref, wt_ref, b_ref, out_ref):
    s = sums_ref[0] + sums_ref[1]
    cnt = jnp.sum(counts_ref[...], axis=0)
    agg = (s + x_ref[...]) / (cnt[:, None] + 1.0)
    acc = jnp.dot(agg, wt_ref[...], preferred_element_type=jnp.float32,
                  precision=jax.lax.Precision.HIGHEST)
    out_ref[...] = jnp.maximum(acc + b_ref[...], 0.0)


def _tc_finish(sums_p, counts_p, x_pad, wt, b2):
    blk = 1024
    grid = N_PAD // blk
    return pl.pallas_call(
        _tc_finish_kernel,
        grid=(grid,),
        in_specs=[
            pl.BlockSpec((NC, blk, D), lambda i: (0, i, 0)),
            pl.BlockSpec((NW, blk), lambda i: (0, i)),
            pl.BlockSpec((blk, D), lambda i: (i, 0)),
            pl.BlockSpec((D, D), lambda i: (0, 0)),
            pl.BlockSpec((1, D), lambda i: (0, 0)),
        ],
        out_specs=pl.BlockSpec((blk, D), lambda i: (i, 0)),
        out_shape=jax.ShapeDtypeStruct((N_PAD, D), jnp.float32),
    )(sums_p, counts_p, x_pad, wt, b2)


def kernel(x, edge_index, W, b):
    src = edge_index[0]
    dst = edge_index[1]
    pad = E_PAD - E
    src_p = jnp.concatenate([src, jnp.zeros((pad,), jnp.int32)])
    dst_p = jnp.concatenate([dst, jnp.full((pad,), N, jnp.int32)])
    src_r = src_p.reshape(NW, NCHUNK, CHUNK)
    dst_r = dst_p.reshape(NW, NCHUNK, CHUNK)
    sums_p, counts_p = _sc_aggregate(x, src_r, dst_r)
    x_pad = jnp.concatenate([x, jnp.zeros((N_PAD - N, D), jnp.float32)])
    out = _tc_finish(sums_p, counts_p, x_pad, W.T, b.reshape(1, D))
    return out[:N]


# speedup vs baseline: 1.0077x; 1.0077x over previous
import functools

import jax
import jax.numpy as jnp
from jax import lax
from jax.experimental import pallas as pl
from jax.experimental.pallas import tpu as pltpu
from jax.experimental.pallas import tpu_sc as plsc

N = 10000
D = 128
E = 320000

NC = 2
NS = 16
NW = NC * NS
CHUNK = 128
NCHUNK = 80
IB = 16
NGROUP = NCHUNK // IB
E_PAD = NW * NCHUNK * CHUNK
N_PAD = 10240
STRIPE = N_PAD // NS
ROWS_PER_TILE_COPY = 128


def _sc_aggregate_kernel(x_hbm, src_hbm, dst_hbm, sums_hbm, counts_hbm,
                         src_v, dst_v, counts_v, gbuf0, srcf_v,
                         sums_acc, sem0, sem1):
    c = lax.axis_index("c")
    s = lax.axis_index("s")
    wid = s * NC + c

    zeros16 = jnp.zeros((16,), jnp.float32)

    def _zrow(i, _):
        for k in range(D // 16):
            gbuf0[i, pl.ds(k * 16, 16)] = zeros16
        return 0
    lax.fori_loop(0, ROWS_PER_TILE_COPY, _zrow, 0)

    def _zcnt(i, _):
        counts_v[pl.ds(i * 16, 16)] = zeros16
        return 0
    lax.fori_loop(0, N_PAD // 16, _zcnt, 0)

    for k in range(STRIPE // ROWS_PER_TILE_COPY):
        pltpu.sync_copy(
            gbuf0,
            sums_acc.at[pl.ds(s * STRIPE + k * ROWS_PER_TILE_COPY,
                              ROWS_PER_TILE_COPY)])

    plsc.subcore_barrier()

    ones16 = jnp.ones((16,), jnp.float32)

    def _half(m, gbuf, sem, refill):
        pltpu.make_async_copy(x_hbm.at[src_v.at[m]], gbuf, sem).wait()
        if refill:
            pltpu.async_copy(x_hbm.at[src_v.at[m + 2]], gbuf, sem)

    pltpu.sync_copy(src_hbm.at[wid], srcf_v)

    def _chunk(m, _):
        pltpu.async_copy(x_hbm.at[srcf_v.at[m]], gbuf0, sem0).wait()
        return 0

    lax.fori_loop(0, NCHUNK, _chunk, 0)

    plsc.subcore_barrier()

    pltpu.sync_copy(sums_acc.at[pl.ds(s * STRIPE, STRIPE)],
                    sums_hbm.at[c, pl.ds(s * STRIPE, STRIPE)])
    pltpu.sync_copy(counts_v, counts_hbm.at[wid])


def _sc_aggregate(x, src_r, dst_r):
    mesh = plsc.VectorSubcoreMesh(core_axis_name="c", subcore_axis_name="s")
    return pl.kernel(
        _sc_aggregate_kernel,
        out_type=(
            jax.ShapeDtypeStruct((NC, N_PAD, D), jnp.float32),
            jax.ShapeDtypeStruct((NW, N_PAD), jnp.float32),
        ),
        mesh=mesh,
        scratch_types=[
            pltpu.VMEM((IB, CHUNK), jnp.int32),
            pltpu.VMEM((IB, CHUNK), jnp.int32),
            pltpu.VMEM((N_PAD,), jnp.float32),
            pltpu.VMEM((CHUNK, D), jnp.float32),
            pltpu.VMEM((NCHUNK, CHUNK), jnp.int32),
            pltpu.VMEM_SHARED((N_PAD, D), jnp.float32),
            pltpu.SemaphoreType.DMA,
            pltpu.SemaphoreType.DMA,
        ],
        compiler_params=pltpu.CompilerParams(needs_layout_passes=False),
    )(x, src_r, dst_r)


def _tc_finish_kernel(sums_ref, counts_ref, x_ref, wt_ref, b_ref, out_ref):
    s = sums_ref[0] + sums_ref[1]
    cnt = jnp.sum(counts_ref[...], axis=0)
    agg = (s + x_ref[...]) / (cnt[:, None] + 1.0)
    acc = jnp.dot(agg, wt_ref[...], preferred_element_type=jnp.float32,
                  precision=jax.lax.Precision.HIGHEST)
    out_ref[...] = jnp.maximum(acc + b_ref[...], 0.0)


def _tc_finish(sums_p, counts_p, x_pad, wt, b2):
    blk = 1024
    grid = N_PAD // blk
    return pl.pallas_call(
        _tc_finish_kernel,
        grid=(grid,),
        in_specs=[
            pl.BlockSpec((NC, blk, D), lambda i: (0, i, 0)),
            pl.BlockSpec((NW, blk), lambda i: (0, i)),
            pl.BlockSpec((blk, D), lambda i: (i, 0)),
            pl.BlockSpec((D, D), lambda i: (0, 0)),
            pl.BlockSpec((1, D), lambda i: (0, 0)),
        ],
        out_specs=pl.BlockSpec((blk, D), lambda i: (i, 0)),
        out_shape=jax.ShapeDtypeStruct((N_PAD, D), jnp.float32),
    )(sums_p, counts_p, x_pad, wt, b2)


def kernel(x, edge_index, W, b):
    src = edge_index[0]
    dst = edge_index[1]
    pad = E_PAD - E
    src_p = jnp.concatenate([src, jnp.zeros((pad,), jnp.int32)])
    dst_p = jnp.concatenate([dst, jnp.full((pad,), N, jnp.int32)])
    src_r = src_p.reshape(NW, NCHUNK, CHUNK)
    dst_r = dst_p.reshape(NW, NCHUNK, CHUNK)
    sums_p, counts_p = _sc_aggregate(x, src_r, dst_r)
    x_pad = jnp.concatenate([x, jnp.zeros((N_PAD - N, D), jnp.float32)])
    out = _tc_finish(sums_p, counts_p, x_pad, W.T, b.reshape(1, D))
    return out[:N]


# speedup vs baseline: 1.3857x; 1.3751x over previous
import jax
import jax.numpy as jnp
from jax import lax
from jax.experimental import pallas as pl
from jax.experimental.pallas import tpu as pltpu
from jax.experimental.pallas import tpu_sc as plsc

N = 10000
D = 128
E = 320000

NC = 2
NS = 16
NW = NC * NS
CHUNK = 128
NCHUNK = -(-E // (NW * CHUNK))
E_PAD = NW * NCHUNK * CHUNK
N_PAD = 10240
STRIPE = N_PAD // NS
ROWS_PER_TILE_COPY = 128


def _sc_aggregate_kernel(x_hbm, src_hbm, dst_hbm, sums_hbm, counts_hbm,
                         src_v, dst_v, counts_v, gbuf0, gbuf1, sums_acc,
                         sem0, sem1):
    c = lax.axis_index("c")
    s = lax.axis_index("s")
    wid = s * NC + c

    pltpu.sync_copy(src_hbm.at[wid], src_v)
    pltpu.sync_copy(dst_hbm.at[wid], dst_v)

    zeros16 = jnp.zeros((16,), jnp.float32)

    def _zrow(i, _):
        for k in range(D // 16):
            gbuf0[i, pl.ds(k * 16, 16)] = zeros16
        return 0
    lax.fori_loop(0, ROWS_PER_TILE_COPY, _zrow, 0)

    def _zcnt(i, _):
        counts_v[pl.ds(i * 16, 16)] = zeros16
        return 0
    lax.fori_loop(0, N_PAD // 16, _zcnt, 0)

    for k in range(STRIPE // ROWS_PER_TILE_COPY):
        pltpu.sync_copy(
            gbuf0,
            sums_acc.at[pl.ds(s * STRIPE + k * ROWS_PER_TILE_COPY,
                              ROWS_PER_TILE_COPY)])

    plsc.subcore_barrier()

    ones16 = jnp.ones((16,), jnp.float32)

    def _chunk(j, _):
        pltpu.async_copy(x_hbm.at[src_v.at[j]], gbuf0, sem0).wait()
        pltpu.sync_copy(gbuf0, sums_acc.at[dst_v.at[j]], add=True)
        for k in range(CHUNK // 16):
            idx = dst_v.at[j][pl.ds(k * 16, 16)]
            plsc.addupdate_scatter(counts_v, [idx], ones16)
        return 0

    lax.fori_loop(0, NCHUNK, _chunk, 0)

    plsc.subcore_barrier()

    pltpu.sync_copy(sums_acc.at[pl.ds(s * STRIPE, STRIPE)],
                    sums_hbm.at[c, pl.ds(s * STRIPE, STRIPE)])
    pltpu.sync_copy(counts_v, counts_hbm.at[wid])


def _sc_aggregate(x, src_r, dst_r):
    mesh = plsc.VectorSubcoreMesh(core_axis_name="c", subcore_axis_name="s")
    return pl.kernel(
        _sc_aggregate_kernel,
        out_type=(
            jax.ShapeDtypeStruct((NC, N_PAD, D), jnp.float32),
            jax.ShapeDtypeStruct((NW, N_PAD), jnp.float32),
        ),
        mesh=mesh,
        scratch_types=[
            pltpu.VMEM((NCHUNK, CHUNK), jnp.int32),
            pltpu.VMEM((NCHUNK, CHUNK), jnp.int32),
            pltpu.VMEM((N_PAD,), jnp.float32),
            pltpu.VMEM((CHUNK, D), jnp.float32),
            pltpu.VMEM((CHUNK, D), jnp.float32),
            pltpu.VMEM_SHARED((N_PAD, D), jnp.float32),
            pltpu.SemaphoreType.DMA,
            pltpu.SemaphoreType.DMA,
        ],
        compiler_params=pltpu.CompilerParams(needs_layout_passes=False),
    )(x, src_r, dst_r)


def _tc_finish_kernel(sums_ref, counts_ref, x_ref, wt_ref, b_ref, out_ref):
    s = sums_ref[0] + sums_ref[1]
    cnt = jnp.sum(counts_ref[...], axis=0)
    agg = (s + x_ref[...]) / (cnt[:, None] + 1.0)
    acc = jnp.dot(agg, wt_ref[...], preferred_element_type=jnp.float32,
                  precision=jax.lax.Precision.HIGHEST)
    out_ref[...] = jnp.maximum(acc + b_ref[...], 0.0)


def _tc_finish(sums_p, counts_p, x_pad, wt, b2):
    blk = 1024
    grid = N_PAD // blk
    return pl.pallas_call(
        _tc_finish_kernel,
        grid=(grid,),
        in_specs=[
            pl.BlockSpec((NC, blk, D), lambda i: (0, i, 0)),
            pl.BlockSpec((NW, blk), lambda i: (0, i)),
            pl.BlockSpec((blk, D), lambda i: (i, 0)),
            pl.BlockSpec((D, D), lambda i: (0, 0)),
            pl.BlockSpec((1, D), lambda i: (0, 0)),
        ],
        out_specs=pl.BlockSpec((blk, D), lambda i: (i, 0)),
        out_shape=jax.ShapeDtypeStruct((N_PAD, D), jnp.float32),
    )(sums_p, counts_p, x_pad, wt, b2)


def kernel(x, edge_index, W, b):
    src = edge_index[0]
    dst = edge_index[1]
    pad = E_PAD - E
    src_p = jnp.concatenate([src, jnp.zeros((pad,), jnp.int32)])
    dst_p = jnp.concatenate([dst, jnp.full((pad,), N, jnp.int32)])
    src_r = src_p.reshape(NW, NCHUNK, CHUNK)
    dst_r = dst_p.reshape(NW, NCHUNK, CHUNK)
    sums_p, counts_p = _sc_aggregate(x, src_r, dst_r)
    x_pad = jnp.concatenate([x, jnp.zeros((N_PAD - N, D), jnp.float32)])
    out = _tc_finish(sums_p, counts_p, x_pad, W.T, b.reshape(1, D))
    return out[:N]


# speedup vs baseline: 1.3864x; 1.0005x over previous
import jax
import jax.numpy as jnp
from jax import lax
from jax.experimental import pallas as pl
from jax.experimental.pallas import tpu as pltpu
from jax.experimental.pallas import tpu_sc as plsc

N = 10000
D = 128
E = 320000

NC = 2
NS = 16
NW = NC * NS
CHUNK = 128
NCHUNK = -(-E // (NW * CHUNK))
E_PAD = NW * NCHUNK * CHUNK
N_PAD = 10240
STRIPE = N_PAD // NS
ROWS_PER_TILE_COPY = 128


def _sc_aggregate_kernel(x_hbm, src_hbm, dst_hbm, sums_hbm, counts_hbm,
                         src_v, dst_v, counts_v, gbuf0, gbuf1, sums_acc,
                         sem0, sem1):
    c = lax.axis_index("c")
    s = lax.axis_index("s")
    wid = s * NC + c

    import contextlib
    with jax.named_scope("phase_stage_idx"):
        pltpu.sync_copy(src_hbm.at[wid], src_v)
        pltpu.sync_copy(dst_hbm.at[wid], dst_v)

    zeros16 = jnp.zeros((16,), jnp.float32)

    with jax.named_scope("phase_zero"):
        def _zrow(i, _):
            for k in range(D // 16):
                gbuf0[i, pl.ds(k * 16, 16)] = zeros16
            return 0
        lax.fori_loop(0, ROWS_PER_TILE_COPY, _zrow, 0)

        def _zcnt(i, _):
            counts_v[pl.ds(i * 16, 16)] = zeros16
            return 0
        lax.fori_loop(0, N_PAD // 16, _zcnt, 0)

        for k in range(STRIPE // ROWS_PER_TILE_COPY):
            pltpu.sync_copy(
                gbuf0,
                sums_acc.at[pl.ds(s * STRIPE + k * ROWS_PER_TILE_COPY,
                                  ROWS_PER_TILE_COPY)])

    with jax.named_scope("phase_barrier1"):
        plsc.subcore_barrier()

    ones16 = jnp.ones((16,), jnp.float32)

    def _chunk(j, _):
        pltpu.async_copy(x_hbm.at[src_v.at[j]], gbuf0, sem0).wait()
        pltpu.sync_copy(gbuf0, sums_acc.at[dst_v.at[j]], add=True)
        for k in range(CHUNK // 16):
            idx = dst_v.at[j][pl.ds(k * 16, 16)]
            plsc.addupdate_scatter(counts_v, [idx], ones16)
        return 0

    with jax.named_scope("phase_mainloop"):
        lax.fori_loop(0, NCHUNK, _chunk, 0)

    with jax.named_scope("phase_barrier2"):
        plsc.subcore_barrier()

    with jax.named_scope("phase_export"):
        pltpu.sync_copy(sums_acc.at[pl.ds(s * STRIPE, STRIPE)],
                        sums_hbm.at[c, pl.ds(s * STRIPE, STRIPE)])
        pltpu.sync_copy(counts_v, counts_hbm.at[wid])


def _sc_aggregate(x, src_r, dst_r):
    mesh = plsc.VectorSubcoreMesh(core_axis_name="c", subcore_axis_name="s")
    return pl.kernel(
        _sc_aggregate_kernel,
        out_type=(
            jax.ShapeDtypeStruct((NC, N_PAD, D), jnp.float32),
            jax.ShapeDtypeStruct((NW, N_PAD), jnp.float32),
        ),
        mesh=mesh,
        scratch_types=[
            pltpu.VMEM((NCHUNK, CHUNK), jnp.int32),
            pltpu.VMEM((NCHUNK, CHUNK), jnp.int32),
            pltpu.VMEM((N_PAD,), jnp.float32),
            pltpu.VMEM((CHUNK, D), jnp.float32),
            pltpu.VMEM((CHUNK, D), jnp.float32),
            pltpu.VMEM_SHARED((N_PAD, D), jnp.float32),
            pltpu.SemaphoreType.DMA,
            pltpu.SemaphoreType.DMA,
        ],
        compiler_params=pltpu.CompilerParams(needs_layout_passes=False),
    )(x, src_r, dst_r)


def _tc_finish_kernel(sums_ref, counts_ref, x_ref, wt_ref, b_ref, out_ref):
    s = sums_ref[0] + sums_ref[1]
    cnt = jnp.sum(counts_ref[...], axis=0)
    agg = (s + x_ref[...]) / (cnt[:, None] + 1.0)
    acc = jnp.dot(agg, wt_ref[...], preferred_element_type=jnp.float32,
                  precision=jax.lax.Precision.HIGHEST)
    out_ref[...] = jnp.maximum(acc + b_ref[...], 0.0)


def _tc_finish(sums_p, counts_p, x_pad, wt, b2):
    blk = 1024
    grid = N_PAD // blk
    return pl.pallas_call(
        _tc_finish_kernel,
        grid=(grid,),
        in_specs=[
            pl.BlockSpec((NC, blk, D), lambda i: (0, i, 0)),
            pl.BlockSpec((NW, blk), lambda i: (0, i)),
            pl.BlockSpec((blk, D), lambda i: (i, 0)),
            pl.BlockSpec((D, D), lambda i: (0, 0)),
            pl.BlockSpec((1, D), lambda i: (0, 0)),
        ],
        out_specs=pl.BlockSpec((blk, D), lambda i: (i, 0)),
        out_shape=jax.ShapeDtypeStruct((N_PAD, D), jnp.float32),
    )(sums_p, counts_p, x_pad, wt, b2)


def kernel(x, edge_index, W, b):
    src = edge_index[0]
    dst = edge_index[1]
    pad = E_PAD - E
    src_p = jnp.concatenate([src, jnp.zeros((pad,), jnp.int32)])
    dst_p = jnp.concatenate([dst, jnp.full((pad,), N, jnp.int32)])
    src_r = src_p.reshape(NW, NCHUNK, CHUNK)
    dst_r = dst_p.reshape(NW, NCHUNK, CHUNK)
    sums_p, counts_p = _sc_aggregate(x, src_r, dst_r)
    x_pad = jnp.concatenate([x, jnp.zeros((N_PAD - N, D), jnp.float32)])
    out = _tc_finish(sums_p, counts_p, x_pad, W.T, b.reshape(1, D))
    return out[:N]


# speedup vs baseline: 2.4506x; 1.7676x over previous
import jax
import jax.numpy as jnp
from jax import lax
from jax.experimental import pallas as pl
from jax.experimental.pallas import tpu as pltpu
from jax.experimental.pallas import tpu_sc as plsc

N = 10000
D = 128
E = 320000

NC = 2
NS = 16
NW = NC * NS
CHUNK = 128
NCHUNK = -(-E // (NW * CHUNK))
E_PAD = NW * NCHUNK * CHUNK
N_PAD = 10240
STRIPE = N_PAD // NS
ROWS_PER_TILE_COPY = 128


def _sc_aggregate_kernel(x_hbm, src_hbm, dst_hbm, sums_hbm, counts_hbm,
                         src_v, dst_v, counts_v, gbuf0, gbuf1, sums_acc,
                         sem0, sem1):
    c = lax.axis_index("c")
    s = lax.axis_index("s")
    wid = s * NC + c

    import contextlib
    with jax.named_scope("phase_stage_idx"):
        pltpu.sync_copy(src_hbm.at[wid], src_v)
        pltpu.sync_copy(dst_hbm.at[wid], dst_v)

    zeros16 = jnp.zeros((16,), jnp.float32)

    with jax.named_scope("phase_zero"):
        def _zrow(i, _):
            for k in range(D // 16):
                gbuf0[i, pl.ds(k * 16, 16)] = zeros16
            return 0
        lax.fori_loop(0, ROWS_PER_TILE_COPY, _zrow, 0)

        def _zcnt(i, _):
            counts_v[pl.ds(i * 16, 16)] = zeros16
            return 0
        lax.fori_loop(0, N_PAD // 16, _zcnt, 0)

        for k in range(STRIPE // ROWS_PER_TILE_COPY):
            pltpu.sync_copy(
                gbuf0,
                sums_acc.at[pl.ds(s * STRIPE + k * ROWS_PER_TILE_COPY,
                                  ROWS_PER_TILE_COPY)])

    with jax.named_scope("phase_barrier1"):
        plsc.subcore_barrier()

    ones16 = jnp.ones((16,), jnp.float32)

    def _chunk(j, _):
        pltpu.async_copy(x_hbm.at[src_v.at[j]], gbuf0, sem0).wait()
        pltpu.sync_copy(gbuf0, sums_acc.at[dst_v.at[j]], add=True)
        for k in range(CHUNK // 16):
            idx = dst_v.at[j][pl.ds(k * 16, 16)]
            plsc.addupdate_scatter(counts_v, [idx], ones16)
        return 0

    with jax.named_scope("phase_mainloop"):
        lax.fori_loop(0, NCHUNK, _chunk, 0)

    with jax.named_scope("phase_barrier2"):
        plsc.subcore_barrier()

    with jax.named_scope("phase_export"):
        pltpu.sync_copy(sums_acc.at[pl.ds(s * STRIPE, STRIPE)],
                        sums_hbm.at[c, pl.ds(s * STRIPE, STRIPE)])
        pltpu.sync_copy(counts_v, counts_hbm.at[wid])


def _sc_aggregate(x, src_r, dst_r):
    mesh = plsc.VectorSubcoreMesh(core_axis_name="c", subcore_axis_name="s")
    return pl.kernel(
        _sc_aggregate_kernel,
        out_type=(
            jax.ShapeDtypeStruct((NC, N_PAD, D), jnp.float32),
            jax.ShapeDtypeStruct((NW, N_PAD), jnp.float32),
        ),
        mesh=mesh,
        scratch_types=[
            pltpu.VMEM((NCHUNK, CHUNK), jnp.int32),
            pltpu.VMEM((NCHUNK, CHUNK), jnp.int32),
            pltpu.VMEM((N_PAD,), jnp.float32),
            pltpu.VMEM((CHUNK, D), jnp.float32),
            pltpu.VMEM((CHUNK, D), jnp.float32),
            pltpu.VMEM_SHARED((N_PAD, D), jnp.float32),
            pltpu.SemaphoreType.DMA,
            pltpu.SemaphoreType.DMA,
        ],
        compiler_params=pltpu.CompilerParams(needs_layout_passes=False),
    )(x, src_r, dst_r)


def _tc_finish_kernel(sums_ref, counts_ref, x_ref, wt_ref, b_ref, out_ref):
    s = sums_ref[0] + sums_ref[1]
    cnt = jnp.sum(counts_ref[...], axis=0)
    agg = (s + x_ref[...]) / (cnt[:, None] + 1.0)
    acc = jnp.dot(agg, wt_ref[...], preferred_element_type=jnp.float32,
                  precision=jax.lax.Precision.HIGHEST)
    out_ref[...] = jnp.maximum(acc + b_ref[...], 0.0)


def _tc_finish(sums_p, counts_p, x_pad, wt, b2):
    blk = 1024
    grid = N_PAD // blk
    return pl.pallas_call(
        _tc_finish_kernel,
        grid=(grid,),
        in_specs=[
            pl.BlockSpec((NC, blk, D), lambda i: (0, i, 0)),
            pl.BlockSpec((NW, blk), lambda i: (0, i)),
            pl.BlockSpec((blk, D), lambda i: (i, 0)),
            pl.BlockSpec((D, D), lambda i: (0, 0)),
            pl.BlockSpec((1, D), lambda i: (0, 0)),
        ],
        out_specs=pl.BlockSpec((blk, D), lambda i: (i, 0)),
        out_shape=jax.ShapeDtypeStruct((N_PAD, D), jnp.float32),
    )(sums_p, counts_p, x_pad, wt, b2)


def kernel(x, edge_index, W, b):
    src = edge_index[0]
    dst = edge_index[1]
    pad = E_PAD - E
    pad_src = (jnp.arange(pad, dtype=jnp.int32) * 37) % N
    pad_dst = N + (jnp.arange(pad, dtype=jnp.int32) % (N_PAD - N))
    src_p = jnp.concatenate([src, pad_src])
    dst_p = jnp.concatenate([dst, pad_dst])
    src_r = src_p.reshape(NW, NCHUNK, CHUNK)
    dst_r = dst_p.reshape(NW, NCHUNK, CHUNK)
    sums_p, counts_p = _sc_aggregate(x, src_r, dst_r)
    x_pad = jnp.concatenate([x, jnp.zeros((N_PAD - N, D), jnp.float32)])
    out = _tc_finish(sums_p, counts_p, x_pad, W.T, b.reshape(1, D))
    return out[:N]


# speedup vs baseline: 3.3953x; 1.3855x over previous
import jax
import jax.numpy as jnp
from jax import lax
from jax.experimental import pallas as pl
from jax.experimental.pallas import tpu as pltpu
from jax.experimental.pallas import tpu_sc as plsc

N = 10000
D = 128
E = 320000

NC = 2
NS = 16
NW = NC * NS
CHUNK = 128
NCHUNK = 80
IB = 16
NGROUP = NCHUNK // IB
E_PAD = NW * NCHUNK * CHUNK
N_PAD = 10240
STRIPE = N_PAD // NS
ROWS_PER_TILE_COPY = 128


def _sc_aggregate_kernel(x_hbm, src_hbm, dst_hbm, sums_hbm, counts_hbm,
                         src_v, dst_v, counts_v, gbuf0, gbuf1, sums_acc,
                         sem0, sem1):
    c = lax.axis_index("c")
    s = lax.axis_index("s")
    wid = s * NC + c

    zeros16 = jnp.zeros((16,), jnp.float32)

    with jax.named_scope("phase_zero"):
        def _zrow(i, _):
            for k in range(D // 16):
                gbuf0[i, pl.ds(k * 16, 16)] = zeros16
            return 0
        lax.fori_loop(0, ROWS_PER_TILE_COPY, _zrow, 0)

        def _zcnt(i, _):
            counts_v[pl.ds(i * 16, 16)] = zeros16
            return 0
        lax.fori_loop(0, N_PAD // 16, _zcnt, 0)

        for k in range(STRIPE // ROWS_PER_TILE_COPY):
            pltpu.sync_copy(
                gbuf0,
                sums_acc.at[pl.ds(s * STRIPE + k * ROWS_PER_TILE_COPY,
                                  ROWS_PER_TILE_COPY)])

    with jax.named_scope("phase_barrier1"):
        plsc.subcore_barrier()

    ones16 = jnp.ones((16,), jnp.float32)

    def _half(m, gbuf, sem, refill):
        pltpu.make_async_copy(x_hbm.at[src_v.at[m]], gbuf, sem).wait()
        pltpu.sync_copy(gbuf, sums_acc.at[dst_v.at[m]], add=True)
        if refill:
            pltpu.async_copy(x_hbm.at[src_v.at[m + 2]], gbuf, sem)
        for k in range(CHUNK // 16):
            idx = dst_v.at[m][pl.ds(k * 16, 16)]
            plsc.addupdate_scatter(counts_v, [idx], ones16)

    def _group(g, _):
        pltpu.sync_copy(src_hbm.at[wid, pl.ds(g * IB, IB)], src_v)
        pltpu.sync_copy(dst_hbm.at[wid, pl.ds(g * IB, IB)], dst_v)
        pltpu.async_copy(x_hbm.at[src_v.at[0]], gbuf0, sem0)
        pltpu.async_copy(x_hbm.at[src_v.at[1]], gbuf1, sem1)

        def _pair(i, _):
            _half(2 * i, gbuf0, sem0, True)
            _half(2 * i + 1, gbuf1, sem1, True)
            return 0

        lax.fori_loop(0, IB // 2 - 1, _pair, 0)
        _half(IB - 2, gbuf0, sem0, False)
        _half(IB - 1, gbuf1, sem1, False)
        return 0

    with jax.named_scope("phase_mainloop"):
        lax.fori_loop(0, NGROUP, _group, 0)

    with jax.named_scope("phase_barrier2"):
        plsc.subcore_barrier()

    with jax.named_scope("phase_export"):
        pltpu.sync_copy(sums_acc.at[pl.ds(s * STRIPE, STRIPE)],
                        sums_hbm.at[c, pl.ds(s * STRIPE, STRIPE)])
        pltpu.sync_copy(counts_v, counts_hbm.at[wid])


def _sc_aggregate(x, src_r, dst_r):
    mesh = plsc.VectorSubcoreMesh(core_axis_name="c", subcore_axis_name="s")
    return pl.kernel(
        _sc_aggregate_kernel,
        out_type=(
            jax.ShapeDtypeStruct((NC, N_PAD, D), jnp.float32),
            jax.ShapeDtypeStruct((NW, N_PAD), jnp.float32),
        ),
        mesh=mesh,
        scratch_types=[
            pltpu.VMEM((IB, CHUNK), jnp.int32),
            pltpu.VMEM((IB, CHUNK), jnp.int32),
            pltpu.VMEM((N_PAD,), jnp.float32),
            pltpu.VMEM((CHUNK, D), jnp.float32),
            pltpu.VMEM((CHUNK, D), jnp.float32),
            pltpu.VMEM_SHARED((N_PAD, D), jnp.float32),
            pltpu.SemaphoreType.DMA,
            pltpu.SemaphoreType.DMA,
        ],
        compiler_params=pltpu.CompilerParams(needs_layout_passes=False),
    )(x, src_r, dst_r)


def _tc_finish_kernel(sums_ref, counts_ref, x_ref, wt_ref, b_ref, out_ref):
    s = sums_ref[0] + sums_ref[1]
    cnt = jnp.sum(counts_ref[...], axis=0)
    agg = (s + x_ref[...]) / (cnt[:, None] + 1.0)
    acc = jnp.dot(agg, wt_ref[...], preferred_element_type=jnp.float32,
                  precision=jax.lax.Precision.HIGHEST)
    out_ref[...] = jnp.maximum(acc + b_ref[...], 0.0)


def _tc_finish(sums_p, counts_p, x_pad, wt, b2):
    blk = 1024
    grid = N_PAD // blk
    return pl.pallas_call(
        _tc_finish_kernel,
        grid=(grid,),
        in_specs=[
            pl.BlockSpec((NC, blk, D), lambda i: (0, i, 0)),
            pl.BlockSpec((NW, blk), lambda i: (0, i)),
            pl.BlockSpec((blk, D), lambda i: (i, 0)),
            pl.BlockSpec((D, D), lambda i: (0, 0)),
            pl.BlockSpec((1, D), lambda i: (0, 0)),
        ],
        out_specs=pl.BlockSpec((blk, D), lambda i: (i, 0)),
        out_shape=jax.ShapeDtypeStruct((N_PAD, D), jnp.float32),
    )(sums_p, counts_p, x_pad, wt, b2)


def kernel(x, edge_index, W, b):
    src = edge_index[0]
    dst = edge_index[1]
    pad = E_PAD - E
    pad_src = (jnp.arange(pad, dtype=jnp.int32) * 37) % N
    pad_dst = N + (jnp.arange(pad, dtype=jnp.int32) % (N_PAD - N))
    src_p = jnp.concatenate([src, pad_src])
    dst_p = jnp.concatenate([dst, pad_dst])
    src_r = src_p.reshape(NW, NCHUNK, CHUNK)
    dst_r = dst_p.reshape(NW, NCHUNK, CHUNK)
    sums_p, counts_p = _sc_aggregate(x, src_r, dst_r)
    x_pad = jnp.concatenate([x, jnp.zeros((N_PAD - N, D), jnp.float32)])
    out = _tc_finish(sums_p, counts_p, x_pad, W.T, b.reshape(1, D))
    return out[:N]


# speedup vs baseline: 3.5352x; 1.0412x over previous
import jax
import jax.numpy as jnp
import numpy as np
from jax import lax
from jax.experimental import pallas as pl
from jax.experimental.pallas import tpu as pltpu
from jax.experimental.pallas import tpu_sc as plsc

N = 10000
D = 128
E = 320000

NC = 2
NS = 16
NW = NC * NS
CHUNK = 128
NCHUNK = 80
IB = 16
NGROUP = NCHUNK // IB
E_PAD = NW * NCHUNK * CHUNK
N_PAD = 10240
STRIPE = N_PAD // NS
ROWS_PER_TILE_COPY = 128


def _sc_aggregate_kernel(x_hbm, src_hbm, dst_hbm, sums_hbm, counts_hbm,
                         src_v, dst_v, counts_v, gbuf0, gbuf1, sums_acc,
                         sem0, sem1):
    c = lax.axis_index("c")
    s = lax.axis_index("s")
    wid = s * NC + c

    zeros16 = jnp.zeros((16,), jnp.float32)

    with jax.named_scope("phase_zero"):
        def _zrow(i, _):
            for k in range(D // 16):
                gbuf0[i, pl.ds(k * 16, 16)] = zeros16
            return 0
        lax.fori_loop(0, ROWS_PER_TILE_COPY, _zrow, 0)

        def _zcnt(i, _):
            counts_v[pl.ds(i * 16, 16)] = zeros16
            return 0
        lax.fori_loop(0, N_PAD // 16, _zcnt, 0)

        for k in range(STRIPE // ROWS_PER_TILE_COPY):
            pltpu.sync_copy(
                gbuf0,
                sums_acc.at[pl.ds(s * STRIPE + k * ROWS_PER_TILE_COPY,
                                  ROWS_PER_TILE_COPY)])

    with jax.named_scope("phase_barrier1"):
        plsc.subcore_barrier()

    ones16 = jnp.ones((16,), jnp.float32)

    def _half(m, gbuf, sem, refill):
        pltpu.make_async_copy(x_hbm.at[src_v.at[m]], gbuf, sem).wait()
        pltpu.sync_copy(gbuf, sums_acc.at[dst_v.at[m]], add=True)
        if refill:
            pltpu.async_copy(x_hbm.at[src_v.at[m + 2]], gbuf, sem)
        for k in range(CHUNK // 16):
            idx = dst_v.at[m][pl.ds(k * 16, 16)]
            plsc.addupdate_scatter(counts_v, [idx], ones16)

    def _group(g, _):
        pltpu.sync_copy(src_hbm.at[wid, pl.ds(g * IB, IB)], src_v)
        pltpu.sync_copy(dst_hbm.at[wid, pl.ds(g * IB, IB)], dst_v)
        pltpu.async_copy(x_hbm.at[src_v.at[0]], gbuf0, sem0)
        pltpu.async_copy(x_hbm.at[src_v.at[1]], gbuf1, sem1)

        def _pair(i, _):
            _half(2 * i, gbuf0, sem0, True)
            _half(2 * i + 1, gbuf1, sem1, True)
            return 0

        lax.fori_loop(0, IB // 2 - 1, _pair, 0)
        _half(IB - 2, gbuf0, sem0, False)
        _half(IB - 1, gbuf1, sem1, False)
        return 0

    with jax.named_scope("phase_mainloop"):
        lax.fori_loop(0, NGROUP, _group, 0)

    with jax.named_scope("phase_barrier2"):
        plsc.subcore_barrier()

    with jax.named_scope("phase_export"):
        pltpu.sync_copy(sums_acc.at[pl.ds(s * STRIPE, STRIPE)],
                        sums_hbm.at[c, pl.ds(s * STRIPE, STRIPE)])
        pltpu.sync_copy(counts_v, counts_hbm.at[wid])


def _sc_aggregate(x, src_r, dst_r):
    mesh = plsc.VectorSubcoreMesh(core_axis_name="c", subcore_axis_name="s")
    return pl.kernel(
        _sc_aggregate_kernel,
        out_type=(
            jax.ShapeDtypeStruct((NC, N_PAD, D), jnp.float32),
            jax.ShapeDtypeStruct((NW, N_PAD), jnp.float32),
        ),
        mesh=mesh,
        scratch_types=[
            pltpu.VMEM((IB, CHUNK), jnp.int32),
            pltpu.VMEM((IB, CHUNK), jnp.int32),
            pltpu.VMEM((N_PAD,), jnp.float32),
            pltpu.VMEM((CHUNK, D), jnp.float32),
            pltpu.VMEM((CHUNK, D), jnp.float32),
            pltpu.VMEM_SHARED((N_PAD, D), jnp.float32),
            pltpu.SemaphoreType.DMA,
            pltpu.SemaphoreType.DMA,
        ],
        compiler_params=pltpu.CompilerParams(needs_layout_passes=False),
    )(x, src_r, dst_r)


def _tc_finish_kernel(sums_ref, counts_ref, x_ref, wt_ref, b_ref, out_ref):
    s = sums_ref[0] + sums_ref[1]
    cnt = jnp.sum(counts_ref[...], axis=0)
    agg = (s + x_ref[...]) / (cnt[:, None] + 1.0)
    acc = jnp.dot(agg, wt_ref[...], preferred_element_type=jnp.float32)
    out_ref[...] = jnp.maximum(acc + b_ref[...], 0.0)


def _tc_finish(sums_p, counts_p, x, wt, b2):
    blk = 1024
    grid = N_PAD // blk
    return pl.pallas_call(
        _tc_finish_kernel,
        grid=(grid,),
        in_specs=[
            pl.BlockSpec((NC, blk, D), lambda i: (0, i, 0)),
            pl.BlockSpec((NW, blk), lambda i: (0, i)),
            pl.BlockSpec((blk, D), lambda i: (i, 0)),
            pl.BlockSpec((D, D), lambda i: (0, 0)),
            pl.BlockSpec((1, D), lambda i: (0, 0)),
        ],
        out_specs=pl.BlockSpec((blk, D), lambda i: (i, 0)),
        out_shape=jax.ShapeDtypeStruct((N, D), jnp.float32),
    )(sums_p, counts_p, x, wt, b2)


def kernel(x, edge_index, W, b):
    src = edge_index[0]
    dst = edge_index[1]
    pad = E_PAD - E
    pad_src = jnp.asarray((np.arange(pad) * 37) % N, dtype=jnp.int32)
    pad_dst = jnp.asarray(N + (np.arange(pad) % (N_PAD - N)), dtype=jnp.int32)
    src_p = jnp.concatenate([src, pad_src])
    dst_p = jnp.concatenate([dst, pad_dst])
    src_r = src_p.reshape(NW, NCHUNK, CHUNK)
    dst_r = dst_p.reshape(NW, NCHUNK, CHUNK)
    sums_p, counts_p = _sc_aggregate(x, src_r, dst_r)
    return _tc_finish(sums_p, counts_p, x, W.T, b.reshape(1, D))


# speedup vs baseline: 3.5473x; 1.0034x over previous
import jax
import jax.numpy as jnp
import numpy as np
from jax import lax
from jax.experimental import pallas as pl
from jax.experimental.pallas import tpu as pltpu
from jax.experimental.pallas import tpu_sc as plsc

N = 10000
D = 128
E = 320000

NC = 2
NS = 16
NW = NC * NS
CHUNK = 128
NCHUNK = 80
IB = 16
NGROUP = NCHUNK // IB
E_PAD = NW * NCHUNK * CHUNK
N_PAD = 10240
STRIPE = N_PAD // NS
ROWS_PER_TILE_COPY = 128


def _sc_aggregate_kernel(x_hbm, src_hbm, dst_hbm, sums_hbm, counts_hbm,
                         src_v, dst_v, counts_v, gbuf0, gbuf1,
                         sums_acc, sem0, sem1):
    c = lax.axis_index("c")
    s = lax.axis_index("s")
    wid = s * NC + c

    with jax.named_scope("phase_stage"):
        pltpu.sync_copy(src_hbm.at[wid, pl.ds(0, IB)], src_v)
        pltpu.sync_copy(dst_hbm.at[wid, pl.ds(0, IB)], dst_v)
        pltpu.async_copy(x_hbm.at[src_v.at[1]], gbuf1, sem1)

    zeros16 = jnp.zeros((16,), jnp.float32)

    with jax.named_scope("phase_zero"):
        def _zrow(i, _):
            for k in range(D // 16):
                gbuf0[i, pl.ds(k * 16, 16)] = zeros16
            return 0
        lax.fori_loop(0, ROWS_PER_TILE_COPY, _zrow, 0)

        def _zcnt(i, _):
            counts_v[pl.ds(i * 16, 16)] = zeros16
            return 0
        lax.fori_loop(0, N_PAD // 16, _zcnt, 0)

        for k in range(STRIPE // ROWS_PER_TILE_COPY):
            pltpu.sync_copy(
                gbuf0,
                sums_acc.at[pl.ds(s * STRIPE + k * ROWS_PER_TILE_COPY,
                                  ROWS_PER_TILE_COPY)])

    pltpu.async_copy(x_hbm.at[src_v.at[0]], gbuf0, sem0)

    with jax.named_scope("phase_barrier1"):
        plsc.subcore_barrier()

    ones16 = jnp.ones((16,), jnp.float32)

    def _half(m, gbuf, sem, refill):
        pltpu.make_async_copy(x_hbm.at[src_v.at[m]], gbuf, sem).wait()
        pltpu.sync_copy(gbuf, sums_acc.at[dst_v.at[m]], add=True)
        if refill:
            pltpu.async_copy(x_hbm.at[src_v.at[m + 2]], gbuf, sem)
        for k in range(CHUNK // 16):
            idx = dst_v.at[m][pl.ds(k * 16, 16)]
            plsc.addupdate_scatter(counts_v, [idx], ones16)

    def _group(g, _):
        def _pair(i, _):
            _half(2 * i, gbuf0, sem0, True)
            _half(2 * i + 1, gbuf1, sem1, True)
            return 0

        lax.fori_loop(0, IB // 2 - 1, _pair, 0)

        @pl.when(g + 1 < NGROUP)
        def _():
            _half(IB - 2, gbuf0, sem0, False)
            _half(IB - 1, gbuf1, sem1, False)
            pltpu.sync_copy(src_hbm.at[wid, pl.ds((g + 1) * IB, IB)], src_v)
            pltpu.sync_copy(dst_hbm.at[wid, pl.ds((g + 1) * IB, IB)], dst_v)
            pltpu.async_copy(x_hbm.at[src_v.at[0]], gbuf0, sem0)
            pltpu.async_copy(x_hbm.at[src_v.at[1]], gbuf1, sem1)

        @pl.when(g + 1 >= NGROUP)
        def _():
            _half(IB - 2, gbuf0, sem0, False)
            _half(IB - 1, gbuf1, sem1, False)
        return 0

    with jax.named_scope("phase_mainloop"):
        lax.fori_loop(0, NGROUP, _group, 0)

    with jax.named_scope("phase_barrier2"):
        plsc.subcore_barrier()

    with jax.named_scope("phase_export"):
        pltpu.sync_copy(sums_acc.at[pl.ds(s * STRIPE, STRIPE)],
                        sums_hbm.at[c, pl.ds(s * STRIPE, STRIPE)])
        pltpu.sync_copy(counts_v, counts_hbm.at[wid])


def _sc_aggregate(x, src_r, dst_r):
    mesh = plsc.VectorSubcoreMesh(core_axis_name="c", subcore_axis_name="s")
    return pl.kernel(
        _sc_aggregate_kernel,
        out_type=(
            jax.ShapeDtypeStruct((NC, N_PAD, D), jnp.float32),
            jax.ShapeDtypeStruct((NW, N_PAD), jnp.float32),
        ),
        mesh=mesh,
        scratch_types=[
            pltpu.VMEM((IB, CHUNK), jnp.int32),
            pltpu.VMEM((IB, CHUNK), jnp.int32),
            pltpu.VMEM((N_PAD,), jnp.float32),
            pltpu.VMEM((CHUNK, D), jnp.float32),
            pltpu.VMEM((CHUNK, D), jnp.float32),
            pltpu.VMEM_SHARED((N_PAD, D), jnp.float32),
            pltpu.SemaphoreType.DMA,
            pltpu.SemaphoreType.DMA,
        ],
        compiler_params=pltpu.CompilerParams(needs_layout_passes=False),
    )(x, src_r, dst_r)


def _tc_finish_kernel(sums_ref, counts_ref, x_ref, wt_ref, b_ref, out_ref):
    s = sums_ref[0] + sums_ref[1]
    cnt = jnp.sum(counts_ref[...], axis=0)
    agg = (s + x_ref[...]) / (cnt[:, None] + 1.0)
    acc = jnp.dot(agg, wt_ref[...], preferred_element_type=jnp.float32)
    out_ref[...] = jnp.maximum(acc + b_ref[...], 0.0)


def _tc_finish(sums_p, counts_p, x, wt, b2):
    blk = 1024
    grid = N_PAD // blk
    return pl.pallas_call(
        _tc_finish_kernel,
        grid=(grid,),
        in_specs=[
            pl.BlockSpec((NC, blk, D), lambda i: (0, i, 0)),
            pl.BlockSpec((NW, blk), lambda i: (0, i)),
            pl.BlockSpec((blk, D), lambda i: (i, 0)),
            pl.BlockSpec((D, D), lambda i: (0, 0)),
            pl.BlockSpec((1, D), lambda i: (0, 0)),
        ],
        out_specs=pl.BlockSpec((blk, D), lambda i: (i, 0)),
        out_shape=jax.ShapeDtypeStruct((N, D), jnp.float32),
    )(sums_p, counts_p, x, wt, b2)


def kernel(x, edge_index, W, b):
    src = edge_index[0]
    dst = edge_index[1]
    pad = E_PAD - E
    pad_src = jnp.asarray((np.arange(pad) * 37) % N, dtype=jnp.int32)
    pad_dst = jnp.asarray(N + (np.arange(pad) % (N_PAD - N)), dtype=jnp.int32)
    src_p = jnp.concatenate([src, pad_src])
    dst_p = jnp.concatenate([dst, pad_dst])
    src_r = src_p.reshape(NW, NCHUNK, CHUNK)
    dst_r = dst_p.reshape(NW, NCHUNK, CHUNK)
    sums_p, counts_p = _sc_aggregate(x, src_r, dst_r)
    return _tc_finish(sums_p, counts_p, x, W.T, b.reshape(1, D))


# speedup vs baseline: 3.7446x; 1.0556x over previous
import jax
import jax.numpy as jnp
import numpy as np
from jax import lax
from jax.experimental import pallas as pl
from jax.experimental.pallas import tpu as pltpu
from jax.experimental.pallas import tpu_sc as plsc

N = 10000
D = 128
E = 320000

NC = 2
NS = 16
NW = NC * NS
CHUNK = 128
NCHUNK = 80
IB = 16
NGROUP = NCHUNK // IB
EPW = E // NW
PPW = NCHUNK * CHUNK - EPW
N_PAD = 10240
STRIPE = N_PAD // NS
ROWS_PER_TILE_COPY = 128


def _sc_aggregate_kernel(x_hbm, edges_hbm, sums_hbm, counts_hbm,
                         src_v, dst_v, counts_v, gbuf0, gbuf1,
                         sums_acc, sem0, sem1):
    c = lax.axis_index("c")
    s = lax.axis_index("s")
    wid = s * NC + c

    with jax.named_scope("phase_stage"):
        pltpu.sync_copy(edges_hbm.at[0, wid, pl.ds(0, IB)], src_v)
        pltpu.sync_copy(edges_hbm.at[1, wid, pl.ds(0, IB)], dst_v)
        pltpu.async_copy(x_hbm.at[src_v.at[1]], gbuf1, sem1)

    zeros16 = jnp.zeros((16,), jnp.float32)

    with jax.named_scope("phase_zero"):
        def _zrow(i, _):
            for k in range(D // 16):
                gbuf0[i, pl.ds(k * 16, 16)] = zeros16
            return 0
        lax.fori_loop(0, ROWS_PER_TILE_COPY, _zrow, 0)

        def _zcnt(i, _):
            counts_v[pl.ds(i * 16, 16)] = zeros16
            return 0
        lax.fori_loop(0, N_PAD // 16, _zcnt, 0)

        for k in range(STRIPE // ROWS_PER_TILE_COPY):
            pltpu.sync_copy(
                gbuf0,
                sums_acc.at[pl.ds(s * STRIPE + k * ROWS_PER_TILE_COPY,
                                  ROWS_PER_TILE_COPY)])

    pltpu.async_copy(x_hbm.at[src_v.at[0]], gbuf0, sem0)

    with jax.named_scope("phase_barrier1"):
        plsc.subcore_barrier()

    ones16 = jnp.ones((16,), jnp.float32)

    def _half(m, gbuf, sem, refill):
        pltpu.make_async_copy(x_hbm.at[src_v.at[m]], gbuf, sem).wait()
        pltpu.sync_copy(gbuf, sums_acc.at[dst_v.at[m]], add=True)
        if refill:
            pltpu.async_copy(x_hbm.at[src_v.at[m + 2]], gbuf, sem)
        for k in range(CHUNK // 16):
            idx = dst_v.at[m][pl.ds(k * 16, 16)]
            plsc.addupdate_scatter(counts_v, [idx], ones16)

    def _group(g, _):
        def _pair(i, _):
            _half(2 * i, gbuf0, sem0, True)
            _half(2 * i + 1, gbuf1, sem1, True)
            return 0

        lax.fori_loop(0, IB // 2 - 1, _pair, 0)

        @pl.when(g + 1 < NGROUP)
        def _():
            _half(IB - 2, gbuf0, sem0, False)
            _half(IB - 1, gbuf1, sem1, False)
            pltpu.sync_copy(edges_hbm.at[0, wid, pl.ds((g + 1) * IB, IB)], src_v)
            pltpu.sync_copy(edges_hbm.at[1, wid, pl.ds((g + 1) * IB, IB)], dst_v)
            pltpu.async_copy(x_hbm.at[src_v.at[0]], gbuf0, sem0)
            pltpu.async_copy(x_hbm.at[src_v.at[1]], gbuf1, sem1)

        @pl.when(g + 1 >= NGROUP)
        def _():
            _half(IB - 2, gbuf0, sem0, False)
            _half(IB - 1, gbuf1, sem1, False)
        return 0

    with jax.named_scope("phase_mainloop"):
        lax.fori_loop(0, NGROUP, _group, 0)

    with jax.named_scope("phase_barrier2"):
        plsc.subcore_barrier()

    with jax.named_scope("phase_export"):
        pltpu.sync_copy(sums_acc.at[pl.ds(s * STRIPE, STRIPE)],
                        sums_hbm.at[c, pl.ds(s * STRIPE, STRIPE)])
        pltpu.sync_copy(counts_v, counts_hbm.at[wid])


def _sc_aggregate(x, edges_r):
    mesh = plsc.VectorSubcoreMesh(core_axis_name="c", subcore_axis_name="s")
    return pl.kernel(
        _sc_aggregate_kernel,
        out_type=(
            jax.ShapeDtypeStruct((NC, N_PAD, D), jnp.float32),
            jax.ShapeDtypeStruct((NW, N_PAD), jnp.float32),
        ),
        mesh=mesh,
        scratch_types=[
            pltpu.VMEM((IB, CHUNK), jnp.int32),
            pltpu.VMEM((IB, CHUNK), jnp.int32),
            pltpu.VMEM((N_PAD,), jnp.float32),
            pltpu.VMEM((CHUNK, D), jnp.float32),
            pltpu.VMEM((CHUNK, D), jnp.float32),
            pltpu.VMEM_SHARED((N_PAD, D), jnp.float32),
            pltpu.SemaphoreType.DMA,
            pltpu.SemaphoreType.DMA,
        ],
        compiler_params=pltpu.CompilerParams(needs_layout_passes=False),
    )(x, edges_r)


def _tc_finish_kernel(sums_ref, counts_ref, x_ref, wt_ref, b_ref, out_ref):
    s = sums_ref[0] + sums_ref[1]
    cnt = jnp.sum(counts_ref[...], axis=0)
    agg = (s + x_ref[...]) / (cnt[:, None] + 1.0)
    acc = jnp.dot(agg, wt_ref[...], preferred_element_type=jnp.float32)
    out_ref[...] = jnp.maximum(acc + b_ref[...], 0.0)


def _tc_finish(sums_p, counts_p, x, wt, b2):
    blk = 1024
    grid = N_PAD // blk
    return pl.pallas_call(
        _tc_finish_kernel,
        grid=(grid,),
        in_specs=[
            pl.BlockSpec((NC, blk, D), lambda i: (0, i, 0)),
            pl.BlockSpec((NW, blk), lambda i: (0, i)),
            pl.BlockSpec((blk, D), lambda i: (i, 0)),
            pl.BlockSpec((D, D), lambda i: (0, 0)),
            pl.BlockSpec((1, D), lambda i: (0, 0)),
        ],
        out_specs=pl.BlockSpec((blk, D), lambda i: (i, 0)),
        out_shape=jax.ShapeDtypeStruct((N, D), jnp.float32),
    )(sums_p, counts_p, x, wt, b2)


def kernel(x, edge_index, W, b):
    pad_srcs = (np.arange(PPW) * 41) % N
    pad_dsts = N + np.arange(PPW)
    pad_blk = jnp.asarray(
        np.broadcast_to(np.stack([pad_srcs, pad_dsts])[:, None, :],
                        (2, NW, PPW)), dtype=jnp.int32)
    er = edge_index.reshape(2, NW, EPW)
    edges_r = jnp.concatenate([er, pad_blk], axis=2).reshape(
        2, NW, NCHUNK, CHUNK)
    sums_p, counts_p = _sc_aggregate(x, edges_r)
    return _tc_finish(sums_p, counts_p, x, W.T, b.reshape(1, D))
